# baseline (speedup 1.0000x reference)
import functools

import jax
import jax.numpy as jnp
from jax import lax
from jax.experimental import pallas as pl
from jax.experimental.pallas import tpu as pltpu
from jax.experimental.pallas import tpu_sc as plsc

N = 10000
E = 320000
D = 128
N_PAD = 10240

NC = 2
NS = 16
NW = NC * NS
CHUNK = 128
CPT = -(-(-(-E // (CHUNK * NW))) // 8) * 8
N_CHUNKS = CPT * NW
E_PAD = N_CHUNKS * CHUNK
ROWS_PER_TILE = N_PAD // NS


def _mesh():
    return plsc.VectorSubcoreMesh(core_axis_name="c", subcore_axis_name="s")


def _hist_body(row_hbm, out_hbm, slab, ones_v, zero_v, hist, sem):
    cid = lax.axis_index("c")
    sid = lax.axis_index("s")
    wid = cid * NS + sid

    def zf(i, _):
        zero_v[pl.ds(i * 16, 16)] = jnp.zeros((16,), jnp.float32)
        return 0
    lax.fori_loop(0, ROWS_PER_TILE // 16, zf, 0)

    def of(i, _):
        ones_v[pl.ds(i * 16, 16)] = jnp.ones((16,), jnp.float32)
        return 0
    lax.fori_loop(0, CHUNK // 16, of, 0)

    pltpu.sync_copy(zero_v, hist.at[pl.ds(sid * ROWS_PER_TILE, ROWS_PER_TILE)])
    pltpu.sync_copy(row_hbm.at[pl.ds(wid * CPT, CPT)], slab)
    plsc.subcore_barrier()

    def body(k, _):
        pltpu.sync_copy(ones_v, hist.at[slab.at[k]], add=True)
        return 0
    lax.fori_loop(0, CPT, body, 0)

    plsc.subcore_barrier()
    pltpu.sync_copy(
        hist.at[pl.ds(sid * ROWS_PER_TILE, ROWS_PER_TILE)],
        out_hbm.at[cid, pl.ds(sid * ROWS_PER_TILE, ROWS_PER_TILE)],
    )


@jax.jit
def _histogram(row2d):
    return pl.kernel(
        _hist_body,
        out_type=jax.ShapeDtypeStruct((NC, N_PAD), jnp.float32),
        mesh=_mesh(),
        scratch_types=[
            pltpu.VMEM((CPT, CHUNK), jnp.int32),
            pltpu.VMEM((CHUNK,), jnp.float32),
            pltpu.VMEM((ROWS_PER_TILE,), jnp.float32),
            pltpu.VMEM_SHARED((N_PAD,), jnp.float32),
            pltpu.SemaphoreType.DMA,
        ],
    )(row2d)


def _dense_block(x_ref, w_ref, b_ref, hp_ref, sup_ref, d_ref):
    deg = hp_ref[0] + hp_ref[1]
    d = lax.rsqrt(jnp.maximum(deg, 1.0))
    sup = lax.dot_general(
        x_ref[...], w_ref[...], (((1,), (1,)), ((), ())),
        preferred_element_type=jnp.float32,
    ) + b_ref[...]
    sup_ref[...] = sup * d
    d_ref[...] = d


@jax.jit
def _dense(x_pad, w, b2d, hist2):
    blk = 512
    grid = N_PAD // blk
    return pl.pallas_call(
        _dense_block,
        grid=(grid,),
        in_specs=[
            pl.BlockSpec((blk, D), lambda i: (i, 0)),
            pl.BlockSpec((D, D), lambda i: (0, 0)),
            pl.BlockSpec((1, D), lambda i: (0, 0)),
            pl.BlockSpec((NC, blk, 1), lambda i: (0, i, 0)),
        ],
        out_specs=[
            pl.BlockSpec((blk, D), lambda i: (i, 0)),
            pl.BlockSpec((blk, 1), lambda i: (i, 0)),
        ],
        out_shape=[
            jax.ShapeDtypeStruct((N_PAD, D), jnp.float32),
            jax.ShapeDtypeStruct((N_PAD, 1), jnp.float32),
        ],
    )(x_pad, w, b2d, hist2)


N_PHASE = 2
CPP = CPT // N_PHASE


def _agg_body(sup_hbm, col_hbm, row_hbm, out_hbm,
              col_slab, row_slab, rows_a, rows_b, acc, gsem_a, gsem_b):
    cid = lax.axis_index("c")
    sid = lax.axis_index("s")
    wid = cid * NS + sid

    def zf(i, _):
        def zg(j, _):
            rows_a[i, pl.ds(j * 16, 16)] = jnp.zeros((16,), jnp.float32)
            return 0
        lax.fori_loop(0, D // 16, zg, 0)
        return 0
    lax.fori_loop(0, CHUNK, zf, 0)

    base = sid * ROWS_PER_TILE
    def zc(i, _):
        pltpu.sync_copy(rows_a, acc.at[pl.ds(base + i * CHUNK, CHUNK)])
        return 0
    lax.fori_loop(0, ROWS_PER_TILE // CHUNK, zc, 0)
    plsc.subcore_barrier()

    def gather(k, buf, sem):
        return pltpu.async_copy(sup_hbm.at[col_slab.at[k]], buf, sem)

    def slot(k, buf, sem):
        pltpu.make_async_copy(sup_hbm.at[col_slab.at[k]], buf, sem).wait()
        pltpu.sync_copy(buf, acc.at[row_slab.at[k]], add=True)

    for ph in range(N_PHASE):
        c0 = wid * CPT + ph * CPP
        pltpu.sync_copy(col_hbm.at[pl.ds(c0, CPP)], col_slab)
        pltpu.sync_copy(row_hbm.at[pl.ds(c0, CPP)], row_slab)

        gather(0, rows_a, gsem_a)
        gather(1, rows_b, gsem_b)

        def body(k2, _):
            k = 2 * k2
            slot(k, rows_a, gsem_a)
            gather(k + 2, rows_a, gsem_a)
            slot(k + 1, rows_b, gsem_b)
            gather(k + 3, rows_b, gsem_b)
            return 0
        lax.fori_loop(0, CPP // 2 - 1, body, 0)
        slot(CPP - 2, rows_a, gsem_a)
        slot(CPP - 1, rows_b, gsem_b)

    plsc.subcore_barrier()
    def wb(i, _):
        pltpu.sync_copy(acc.at[pl.ds(base + i * CHUNK, CHUNK)],
                        out_hbm.at[cid, pl.ds(base + i * CHUNK, CHUNK)])
        return 0
    lax.fori_loop(0, ROWS_PER_TILE // CHUNK, wb, 0)


@jax.jit
def _aggregate(sup, col2d, row2d):
    return pl.kernel(
        _agg_body,
        out_type=jax.ShapeDtypeStruct((NC, N_PAD, D), jnp.float32),
        mesh=_mesh(),
        scratch_types=[
            pltpu.VMEM((CPP, CHUNK), jnp.int32),
            pltpu.VMEM((CPP, CHUNK), jnp.int32),
            pltpu.VMEM((CHUNK, D), jnp.float32),
            pltpu.VMEM((CHUNK, D), jnp.float32),
            pltpu.VMEM_SHARED((N_PAD, D), jnp.float32),
            pltpu.SemaphoreType.DMA,
            pltpu.SemaphoreType.DMA,
        ],
    )(sup, col2d, row2d)


def _fin_block(p_ref, d_ref, o_ref):
    o_ref[...] = (p_ref[0] + p_ref[1]) * d_ref[...]


@jax.jit
def _finalize(partials, d):
    blk = 512
    grid = N_PAD // blk
    return pl.pallas_call(
        _fin_block,
        grid=(grid,),
        in_specs=[
            pl.BlockSpec((NC, blk, D), lambda i: (0, i, 0)),
            pl.BlockSpec((blk, 1), lambda i: (i, 0)),
        ],
        out_specs=pl.BlockSpec((blk, D), lambda i: (i, 0)),
        out_shape=jax.ShapeDtypeStruct((N_PAD, D), jnp.float32),
    )(partials, d)


def kernel(node_features, edge_index, W, b):
    pad = jnp.full((2, E_PAD - E), N, dtype=jnp.int32)
    ei = jnp.concatenate([edge_index, pad], axis=1)
    row2d = ei[0].reshape(N_CHUNKS, CHUNK)
    col2d = ei[1].reshape(N_CHUNKS, CHUNK)
    x_pad = jnp.zeros((N_PAD, D), jnp.float32).at[:N].set(node_features)

    hist = _histogram(row2d)
    hist2 = hist[:, :, None]
    sup, d = _dense(x_pad, W, b.reshape(1, D), hist2)
    partials = _aggregate(sup, col2d, row2d)
    out = _finalize(partials, d)
    return out[:N]
